# baseline (speedup 1.0000x reference)
import jax
import jax.numpy as jnp
from jax.experimental import pallas as pl
from jax.experimental.pallas import tpu as pltpu

STYLE = 256
NC = 256
NCOMP = 8
B, H, W = 4, 64, 64
NLAB = 16
EPS = 1e-5


def _prep_kernel(seg_ref, sc_ref, ex_ref, fcwt_ref, fcb_ref, wmid_ref, xt_ref,
                 oh_ref, taps_ref, sums_ref, sqs_ref):
    seg = seg_ref[0]
    mask = (seg != 0).astype(jnp.float32)

    jidx = (jax.lax.broadcasted_iota(jnp.int32, (NCOMP, 1, 1), 0) + 1
            ).astype(jnp.float32)
    lab = jnp.max(mask * jidx, axis=0)

    l_iota = jax.lax.broadcasted_iota(jnp.int32, (1, 1, NLAB), 2
                                      ).astype(jnp.float32)
    oh = (lab[:, :, None] == l_iota).astype(jnp.float32)
    zc = jnp.zeros((H, 1, NLAB), jnp.float32)
    ohp = jnp.concatenate([zc, oh, zc], axis=1)
    zr = jnp.zeros((1, W + 2, NLAB), jnp.float32)
    ohp = jnp.concatenate([zr, ohp, zr], axis=0)
    oh_ref[0] = ohp.astype(jnp.bfloat16)

    sc = sc_ref[0]
    sc_mean = jnp.mean(sc, axis=0, keepdims=True)
    mus = []
    for j in range(NCOMP):
        area = jnp.sum(mask[j])
        code_e = jnp.where(ex_ref[0, 0, j] == 1.0, sc[j:j + 1, :],
                           sc[NCOMP:NCOMP + 1, :])
        code = jnp.where(area > 0.0, code_e, sc_mean)
        mu = jnp.dot(code, fcwt_ref[j], preferred_element_type=jnp.float32)
        mu = jnp.maximum(mu + fcb_ref[j:j + 1, :], 0.0)
        mus.append(mu)
    m16 = jnp.concatenate(
        [jnp.zeros((1, STYLE), jnp.float32)] + mus
        + [jnp.zeros((NLAB - 1 - NCOMP, STYLE), jnp.float32)], axis=0)

    for t in range(9):
        taps_ref[0, t] = jnp.dot(m16, wmid_ref[t],
                                 preferred_element_type=jnp.float32)

    xb = xt_ref[0]
    sums_ref[0, 0] = jnp.sum(xb, axis=(0, 1))
    sqs_ref[0, 0] = jnp.sum(xb * xb, axis=(0, 1))


def _conv_kernel(xt_ref, oh_ref, taps_ref, w2_ref, sums_ref, sqs_ref,
                 bnw_ref, bnb_ref, cgb_ref, cbb_ref, out_ref, scratch_ref):
    k = pl.program_id(1)
    base = k * 8

    n = float(B * H * W)
    mean = jnp.sum(sums_ref[:, 0, :], axis=0).reshape(1, 1, NC) / n
    var = jnp.sum(sqs_ref[:, 0, :], axis=0).reshape(1, 1, NC) / n - mean * mean
    scale = bnw_ref[0].reshape(1, 1, NC) * jax.lax.rsqrt(var + EPS)
    shift = bnb_ref[0].reshape(1, 1, NC) - mean * scale

    xin = xt_ref[0, pl.ds(base, 8), :, :]
    coarse_c = xin * scale + shift
    scratch_ref[1:9, 1:W + 1, :] = coarse_c.astype(jnp.bfloat16)
    top = xt_ref[0, pl.ds(jnp.maximum(base - 1, 0), 1), :, :] * scale + shift
    scratch_ref[0:1, 1:W + 1, :] = jnp.where(k == 0, 0.0, top).astype(jnp.bfloat16)
    bot = xt_ref[0, pl.ds(jnp.minimum(base + 8, H - 1), 1), :, :] * scale + shift
    scratch_ref[9:10, 1:W + 1, :] = jnp.where(k == H // 8 - 1, 0.0,
                                              bot).astype(jnp.bfloat16)
    zcol = jnp.zeros((10, 1, NC), jnp.bfloat16)
    scratch_ref[:, 0:1, :] = zcol
    scratch_ref[:, W + 1:W + 2, :] = zcol

    oh = oh_ref[0, pl.ds(base, 10), :, :]

    acc = jnp.zeros((8 * W, NC), jnp.float32)
    for dx in range(3):
        lhs = scratch_ref[:, dx:dx + W, :]
        ohl = oh[:, dx:dx + W, :]
        for dy in range(3):
            t = dy * 3 + dx
            v = lhs[dy:dy + 8].reshape(8 * W, NC)
            acc += jnp.dot(v, w2_ref[t], preferred_element_type=jnp.float32)
            u = ohl[dy:dy + 8].reshape(8 * W, NLAB)
            acc += jnp.dot(u, taps_ref[0, t], preferred_element_type=jnp.float32)

    center = coarse_c.reshape(8 * W, NC)
    bias = (cgb_ref[0] + cbb_ref[0]).reshape(1, NC)
    out_ref[0] = (acc + center + bias).reshape(8, W, NC)


def kernel(x, segmap, style_codes, exist_codes, fc_w, fc_b,
           conv_gamma_w, conv_gamma_b, conv_beta_w, conv_beta_b,
           bn_weight, bn_bias):
    xt = jnp.zeros((B, H, W, NC), jnp.float32)
    exf = exist_codes.astype(jnp.float32).reshape(B, 1, NCOMP)
    fcwt = jnp.transpose(fc_w, (0, 2, 1))
    wmid = jnp.transpose(conv_gamma_w[:, :NC] + conv_beta_w,
                         (2, 3, 1, 0)).reshape(9, NC, NC)
    w2 = jnp.transpose(conv_gamma_w[:, NC:], (2, 3, 1, 0)).reshape(9, NC, NC)

    oh, taps, sums, sqs = pl.pallas_call(
        _prep_kernel,
        grid=(B,),
        in_specs=[
            pl.BlockSpec((1, NCOMP, H, W), lambda i: (i, 0, 0, 0)),
            pl.BlockSpec((1, NCOMP + 1, STYLE), lambda i: (i, 0, 0)),
            pl.BlockSpec((1, 1, NCOMP), lambda i: (i, 0, 0)),
            pl.BlockSpec((NCOMP, STYLE, STYLE), lambda i: (0, 0, 0)),
            pl.BlockSpec((NCOMP, STYLE), lambda i: (0, 0)),
            pl.BlockSpec((9, NC, NC), lambda i: (0, 0, 0)),
            pl.BlockSpec((1, H, W, NC), lambda i: (i, 0, 0, 0)),
        ],
        out_specs=[
            pl.BlockSpec((1, H + 2, W + 2, NLAB), lambda i: (i, 0, 0, 0)),
            pl.BlockSpec((1, 9, NLAB, NC), lambda i: (i, 0, 0, 0)),
            pl.BlockSpec((1, 1, NC), lambda i: (i, 0, 0)),
            pl.BlockSpec((1, 1, NC), lambda i: (i, 0, 0)),
        ],
        out_shape=[
            jax.ShapeDtypeStruct((B, H + 2, W + 2, NLAB), jnp.bfloat16),
            jax.ShapeDtypeStruct((B, 9, NLAB, NC), jnp.float32),
            jax.ShapeDtypeStruct((B, 1, NC), jnp.float32),
            jax.ShapeDtypeStruct((B, 1, NC), jnp.float32),
        ],
    )(segmap, style_codes, exf, fcwt, fc_b, wmid, xt)

    taps = taps.astype(jnp.bfloat16)
    w2 = w2.astype(jnp.bfloat16)

    out_nhwc = pl.pallas_call(
        _conv_kernel,
        grid=(B, H // 8),
        in_specs=[
            pl.BlockSpec((1, H, W, NC), lambda i, k: (i, 0, 0, 0)),
            pl.BlockSpec((1, H + 2, W + 2, NLAB), lambda i, k: (i, 0, 0, 0)),
            pl.BlockSpec((1, 9, NLAB, NC), lambda i, k: (i, 0, 0, 0)),
            pl.BlockSpec((9, NC, NC), lambda i, k: (0, 0, 0)),
            pl.BlockSpec((B, 1, NC), lambda i, k: (0, 0, 0)),
            pl.BlockSpec((B, 1, NC), lambda i, k: (0, 0, 0)),
            pl.BlockSpec((1, NC), lambda i, k: (0, 0)),
            pl.BlockSpec((1, NC), lambda i, k: (0, 0)),
            pl.BlockSpec((1, NC), lambda i, k: (0, 0)),
            pl.BlockSpec((1, NC), lambda i, k: (0, 0)),
        ],
        out_specs=pl.BlockSpec((1, 8, W, NC), lambda i, k: (i, k, 0, 0)),
        out_shape=jax.ShapeDtypeStruct((B, H, W, NC), jnp.float32),
        scratch_shapes=[pltpu.VMEM((10, W + 2, NC), jnp.bfloat16)],
    )(xt, oh, taps, w2, sums, sqs,
      bn_weight.reshape(1, NC), bn_bias.reshape(1, NC),
      conv_gamma_b.reshape(1, NC), conv_beta_b.reshape(1, NC))

    return out_nhwc


# speedup vs baseline: 1.1738x; 1.1738x over previous
import jax
import jax.numpy as jnp
from jax.experimental import pallas as pl
from jax.experimental.pallas import tpu as pltpu

STYLE = 256
NC = 256
NCOMP = 8
B, H, W = 4, 64, 64
NLAB = 16
EPS = 1e-5


def _prep_kernel(seg_ref, sc_ref, ex_ref, fcwt_ref, fcb_ref, wmid_ref, xt_ref,
                 oh_ref, taps_ref, sums_ref, sqs_ref):
    seg = seg_ref[0]
    mask = (seg != 0).astype(jnp.float32)

    jidx = (jax.lax.broadcasted_iota(jnp.int32, (NCOMP, 1, 1), 0) + 1
            ).astype(jnp.float32)
    lab = jnp.max(mask * jidx, axis=0)

    l_iota = jax.lax.broadcasted_iota(jnp.int32, (1, 1, NLAB), 2
                                      ).astype(jnp.float32)
    oh = (lab[:, :, None] == l_iota).astype(jnp.float32)
    zc = jnp.zeros((H, 1, NLAB), jnp.float32)
    ohp = jnp.concatenate([zc, oh, zc], axis=1)
    zr = jnp.zeros((1, W + 2, NLAB), jnp.float32)
    ohp = jnp.concatenate([zr, ohp, zr], axis=0)
    oh_ref[0] = ohp.astype(jnp.bfloat16)

    sc = sc_ref[0]
    sc_mean = jnp.mean(sc, axis=0, keepdims=True)
    mus = []
    for j in range(NCOMP):
        area = jnp.sum(mask[j])
        code_e = jnp.where(ex_ref[0, 0, j] == 1.0, sc[j:j + 1, :],
                           sc[NCOMP:NCOMP + 1, :])
        code = jnp.where(area > 0.0, code_e, sc_mean)
        mu = jnp.dot(code, fcwt_ref[j], preferred_element_type=jnp.float32)
        mu = jnp.maximum(mu + fcb_ref[j:j + 1, :], 0.0)
        mus.append(mu)
    m16 = jnp.concatenate(
        [jnp.zeros((1, STYLE), jnp.float32)] + mus
        + [jnp.zeros((NLAB - 1 - NCOMP, STYLE), jnp.float32)], axis=0)

    for t in range(9):
        taps_ref[0, t] = jnp.dot(m16, wmid_ref[t],
                                 preferred_element_type=jnp.float32)

    xb = xt_ref[0]
    sums_ref[0, 0] = jnp.sum(xb, axis=(0, 1))
    sqs_ref[0, 0] = jnp.sum(xb * xb, axis=(0, 1))


def _conv_kernel(xt_ref, oh_ref, taps_ref, w2_ref, sums_ref, sqs_ref,
                 bnw_ref, bnb_ref, cgb_ref, cbb_ref, out_ref, scratch_ref):
    n = float(B * H * W)
    mean = jnp.sum(sums_ref[:, 0, :], axis=0).reshape(1, 1, NC) / n
    var = jnp.sum(sqs_ref[:, 0, :], axis=0).reshape(1, 1, NC) / n - mean * mean
    scale = bnw_ref[0].reshape(1, 1, NC) * jax.lax.rsqrt(var + EPS)
    shift = bnb_ref[0].reshape(1, 1, NC) - mean * scale
    bias = (cgb_ref[0] + cbb_ref[0]).reshape(1, NC)

    for k in range(H // 8):
        base = k * 8
        xin = xt_ref[0, base:base + 8, :, :]
        coarse_c = xin * scale + shift
        scratch_ref[1:9, 1:W + 1, :] = coarse_c.astype(jnp.bfloat16)
        if k == 0:
            scratch_ref[0:1, 1:W + 1, :] = jnp.zeros((1, W, NC), jnp.bfloat16)
        else:
            top = xt_ref[0, base - 1:base, :, :] * scale + shift
            scratch_ref[0:1, 1:W + 1, :] = top.astype(jnp.bfloat16)
        if k == H // 8 - 1:
            scratch_ref[9:10, 1:W + 1, :] = jnp.zeros((1, W, NC), jnp.bfloat16)
        else:
            bot = xt_ref[0, base + 8:base + 9, :, :] * scale + shift
            scratch_ref[9:10, 1:W + 1, :] = bot.astype(jnp.bfloat16)
        zcol = jnp.zeros((10, 1, NC), jnp.bfloat16)
        scratch_ref[:, 0:1, :] = zcol
        scratch_ref[:, W + 1:W + 2, :] = zcol

        oh = oh_ref[0, base:base + 10, :, :]

        acc = jnp.zeros((8 * W, NC), jnp.float32)
        for dx in range(3):
            lhs = scratch_ref[:, dx:dx + W, :]
            ohl = oh[:, dx:dx + W, :]
            for dy in range(3):
                t = dy * 3 + dx
                v = lhs[dy:dy + 8].reshape(8 * W, NC)
                acc += jnp.dot(v, w2_ref[t],
                               preferred_element_type=jnp.float32)
                u = ohl[dy:dy + 8].reshape(8 * W, NLAB)
                acc += jnp.dot(u, taps_ref[0, t],
                               preferred_element_type=jnp.float32)

        out_ref[0, base:base + 8] = (acc + coarse_c.reshape(8 * W, NC)
                                     + bias).reshape(8, W, NC)


def kernel(x, segmap, style_codes, exist_codes, fc_w, fc_b,
           conv_gamma_w, conv_gamma_b, conv_beta_w, conv_beta_b,
           bn_weight, bn_bias):
    xt = jnp.transpose(x, (0, 2, 3, 1))
    exf = exist_codes.astype(jnp.float32).reshape(B, 1, NCOMP)
    fcwt = jnp.transpose(fc_w, (0, 2, 1))
    wmid = jnp.transpose(conv_gamma_w[:, :NC] + conv_beta_w,
                         (2, 3, 1, 0)).reshape(9, NC, NC)
    w2 = jnp.transpose(conv_gamma_w[:, NC:], (2, 3, 1, 0)).reshape(9, NC, NC)

    oh, taps, sums, sqs = pl.pallas_call(
        _prep_kernel,
        grid=(B,),
        in_specs=[
            pl.BlockSpec((1, NCOMP, H, W), lambda i: (i, 0, 0, 0)),
            pl.BlockSpec((1, NCOMP + 1, STYLE), lambda i: (i, 0, 0)),
            pl.BlockSpec((1, 1, NCOMP), lambda i: (i, 0, 0)),
            pl.BlockSpec((NCOMP, STYLE, STYLE), lambda i: (0, 0, 0)),
            pl.BlockSpec((NCOMP, STYLE), lambda i: (0, 0)),
            pl.BlockSpec((9, NC, NC), lambda i: (0, 0, 0)),
            pl.BlockSpec((1, H, W, NC), lambda i: (i, 0, 0, 0)),
        ],
        out_specs=[
            pl.BlockSpec((1, H + 2, W + 2, NLAB), lambda i: (i, 0, 0, 0)),
            pl.BlockSpec((1, 9, NLAB, NC), lambda i: (i, 0, 0, 0)),
            pl.BlockSpec((1, 1, NC), lambda i: (i, 0, 0)),
            pl.BlockSpec((1, 1, NC), lambda i: (i, 0, 0)),
        ],
        out_shape=[
            jax.ShapeDtypeStruct((B, H + 2, W + 2, NLAB), jnp.bfloat16),
            jax.ShapeDtypeStruct((B, 9, NLAB, NC), jnp.float32),
            jax.ShapeDtypeStruct((B, 1, NC), jnp.float32),
            jax.ShapeDtypeStruct((B, 1, NC), jnp.float32),
        ],
    )(segmap, style_codes, exf, fcwt, fc_b, wmid, xt)

    taps = taps.astype(jnp.bfloat16)
    w2 = w2.astype(jnp.bfloat16)

    out_nhwc = pl.pallas_call(
        _conv_kernel,
        grid=(B,),
        in_specs=[
            pl.BlockSpec((1, H, W, NC), lambda i: (i, 0, 0, 0)),
            pl.BlockSpec((1, H + 2, W + 2, NLAB), lambda i: (i, 0, 0, 0)),
            pl.BlockSpec((1, 9, NLAB, NC), lambda i: (i, 0, 0, 0)),
            pl.BlockSpec((9, NC, NC), lambda i: (0, 0, 0)),
            pl.BlockSpec((B, 1, NC), lambda i: (0, 0, 0)),
            pl.BlockSpec((B, 1, NC), lambda i: (0, 0, 0)),
            pl.BlockSpec((1, NC), lambda i: (0, 0)),
            pl.BlockSpec((1, NC), lambda i: (0, 0)),
            pl.BlockSpec((1, NC), lambda i: (0, 0)),
            pl.BlockSpec((1, NC), lambda i: (0, 0)),
        ],
        out_specs=pl.BlockSpec((1, H, W, NC), lambda i: (i, 0, 0, 0)),
        out_shape=jax.ShapeDtypeStruct((B, H, W, NC), jnp.float32),
        scratch_shapes=[pltpu.VMEM((10, W + 2, NC), jnp.bfloat16)],
    )(xt, oh, taps, w2, sums, sqs,
      bn_weight.reshape(1, NC), bn_bias.reshape(1, NC),
      conv_gamma_b.reshape(1, NC), conv_beta_b.reshape(1, NC))

    return jnp.transpose(out_nhwc, (0, 3, 1, 2))


# speedup vs baseline: 1.5345x; 1.3073x over previous
import jax
import jax.numpy as jnp
from jax.experimental import pallas as pl
from jax.experimental.pallas import tpu as pltpu

STYLE = 256
NC = 256
NCOMP = 8
B, H, W = 4, 64, 64
NLAB = 16
EPS = 1e-5


def _prep_kernel(seg_ref, sc_ref, ex_ref, fcwt_ref, fcb_ref, wmid_ref, xt_ref,
                 oh_ref, taps_ref, sums_ref, sqs_ref):
    seg = seg_ref[0]
    mask = (seg != 0).astype(jnp.float32)

    jidx = (jax.lax.broadcasted_iota(jnp.int32, (NCOMP, 1, 1), 0) + 1
            ).astype(jnp.float32)
    lab = jnp.max(mask * jidx, axis=0)

    l_iota = jax.lax.broadcasted_iota(jnp.int32, (1, 1, NLAB), 2
                                      ).astype(jnp.float32)
    oh = (lab[:, :, None] == l_iota).astype(jnp.float32)
    zc = jnp.zeros((H, 1, NLAB), jnp.float32)
    ohp = jnp.concatenate([zc, oh, zc], axis=1)
    zr = jnp.zeros((1, W + 2, NLAB), jnp.float32)
    ohp = jnp.concatenate([zr, ohp, zr], axis=0)
    oh_ref[0] = ohp.astype(jnp.bfloat16)

    sc = sc_ref[0]
    sc_mean = jnp.mean(sc, axis=0, keepdims=True)
    mus = []
    for j in range(NCOMP):
        area = jnp.sum(mask[j])
        code_e = jnp.where(ex_ref[0, 0, j] == 1.0, sc[j:j + 1, :],
                           sc[NCOMP:NCOMP + 1, :])
        code = jnp.where(area > 0.0, code_e, sc_mean)
        mu = jnp.dot(code, fcwt_ref[j], preferred_element_type=jnp.float32)
        mu = jnp.maximum(mu + fcb_ref[j:j + 1, :], 0.0)
        mus.append(mu)
    m16 = jnp.concatenate(
        [jnp.zeros((1, STYLE), jnp.float32)] + mus
        + [jnp.zeros((NLAB - 1 - NCOMP, STYLE), jnp.float32)], axis=0)

    for dx in range(3):
        for dy in range(3):
            p = jnp.dot(m16, wmid_ref[dy * 3 + dx],
                        preferred_element_type=jnp.float32)
            taps_ref[0, dx, dy * NLAB:(dy + 1) * NLAB, :] = p.astype(jnp.bfloat16)

    xb = xt_ref[0]
    sums_ref[0, 0] = jnp.sum(xb, axis=(0, 1))
    sqs_ref[0, 0] = jnp.sum(xb * xb, axis=(0, 1))


def _conv_kernel(xt_ref, oh_ref, taps_ref, w2_ref, sums_ref, sqs_ref,
                 bnw_ref, bnb_ref, cgb_ref, cbb_ref, out_ref, scratch_ref):
    n = float(B * H * W)
    mean = jnp.sum(sums_ref[:, 0, :], axis=0).reshape(1, 1, NC) / n
    var = jnp.sum(sqs_ref[:, 0, :], axis=0).reshape(1, 1, NC) / n - mean * mean
    scale = bnw_ref[0].reshape(1, 1, NC) * jax.lax.rsqrt(var + EPS)
    shift = bnb_ref[0].reshape(1, 1, NC) - mean * scale
    bias = (cgb_ref[0] + cbb_ref[0]).reshape(1, NC)

    for k in range(H // 8):
        base = k * 8
        xin = xt_ref[0, base:base + 8, :, :]
        coarse_c = xin * scale + shift
        scratch_ref[1:9, 1:W + 1, :] = coarse_c.astype(jnp.bfloat16)
        if k == 0:
            scratch_ref[0:1, 1:W + 1, :] = jnp.zeros((1, W, NC), jnp.bfloat16)
        else:
            top = xt_ref[0, base - 1:base, :, :] * scale + shift
            scratch_ref[0:1, 1:W + 1, :] = top.astype(jnp.bfloat16)
        if k == H // 8 - 1:
            scratch_ref[9:10, 1:W + 1, :] = jnp.zeros((1, W, NC), jnp.bfloat16)
        else:
            bot = xt_ref[0, base + 8:base + 9, :, :] * scale + shift
            scratch_ref[9:10, 1:W + 1, :] = bot.astype(jnp.bfloat16)
        zcol = jnp.zeros((10, 1, NC), jnp.bfloat16)
        scratch_ref[:, 0:1, :] = zcol
        scratch_ref[:, W + 1:W + 2, :] = zcol

        oh = oh_ref[0, base:base + 10, :, :]

        acc = jnp.zeros((8 * W, NC), jnp.float32)
        for dx in range(3):
            lhs = scratch_ref[:, dx:dx + W, :]
            ohl = oh[:, dx:dx + W, :]
            v3 = jnp.concatenate([lhs[0:8], lhs[1:9], lhs[2:10]],
                                 axis=2).reshape(8 * W, 3 * NC)
            acc += jnp.dot(v3, w2_ref[dx],
                           preferred_element_type=jnp.float32)
            u3 = jnp.concatenate([ohl[0:8], ohl[1:9], ohl[2:10]],
                                 axis=2).reshape(8 * W, 3 * NLAB)
            acc += jnp.dot(u3, taps_ref[0, dx],
                           preferred_element_type=jnp.float32)

        out_ref[0, base:base + 8] = (acc + coarse_c.reshape(8 * W, NC)
                                     + bias).reshape(8, W, NC)


def kernel(x, segmap, style_codes, exist_codes, fc_w, fc_b,
           conv_gamma_w, conv_gamma_b, conv_beta_w, conv_beta_b,
           bn_weight, bn_bias):
    xt = jnp.transpose(x, (0, 2, 3, 1))
    exf = exist_codes.astype(jnp.float32).reshape(B, 1, NCOMP)
    fcwt = jnp.transpose(fc_w, (0, 2, 1))
    wmid = jnp.transpose(conv_gamma_w[:, :NC] + conv_beta_w,
                         (2, 3, 1, 0)).reshape(9, NC, NC)
    w2 = jnp.transpose(conv_gamma_w[:, NC:],
                       (3, 2, 1, 0)).reshape(3, 3 * NC, NC).astype(jnp.bfloat16)

    oh, taps, sums, sqs = pl.pallas_call(
        _prep_kernel,
        grid=(B,),
        in_specs=[
            pl.BlockSpec((1, NCOMP, H, W), lambda i: (i, 0, 0, 0)),
            pl.BlockSpec((1, NCOMP + 1, STYLE), lambda i: (i, 0, 0)),
            pl.BlockSpec((1, 1, NCOMP), lambda i: (i, 0, 0)),
            pl.BlockSpec((NCOMP, STYLE, STYLE), lambda i: (0, 0, 0)),
            pl.BlockSpec((NCOMP, STYLE), lambda i: (0, 0)),
            pl.BlockSpec((9, NC, NC), lambda i: (0, 0, 0)),
            pl.BlockSpec((1, H, W, NC), lambda i: (i, 0, 0, 0)),
        ],
        out_specs=[
            pl.BlockSpec((1, H + 2, W + 2, NLAB), lambda i: (i, 0, 0, 0)),
            pl.BlockSpec((1, 3, 3 * NLAB, NC), lambda i: (i, 0, 0, 0)),
            pl.BlockSpec((1, 1, NC), lambda i: (i, 0, 0)),
            pl.BlockSpec((1, 1, NC), lambda i: (i, 0, 0)),
        ],
        out_shape=[
            jax.ShapeDtypeStruct((B, H + 2, W + 2, NLAB), jnp.bfloat16),
            jax.ShapeDtypeStruct((B, 3, 3 * NLAB, NC), jnp.bfloat16),
            jax.ShapeDtypeStruct((B, 1, NC), jnp.float32),
            jax.ShapeDtypeStruct((B, 1, NC), jnp.float32),
        ],
    )(segmap, style_codes, exf, fcwt, fc_b, wmid, xt)

    out_nhwc = pl.pallas_call(
        _conv_kernel,
        grid=(B,),
        in_specs=[
            pl.BlockSpec((1, H, W, NC), lambda i: (i, 0, 0, 0)),
            pl.BlockSpec((1, H + 2, W + 2, NLAB), lambda i: (i, 0, 0, 0)),
            pl.BlockSpec((1, 3, 3 * NLAB, NC), lambda i: (i, 0, 0, 0)),
            pl.BlockSpec((3, 3 * NC, NC), lambda i: (0, 0, 0)),
            pl.BlockSpec((B, 1, NC), lambda i: (0, 0, 0)),
            pl.BlockSpec((B, 1, NC), lambda i: (0, 0, 0)),
            pl.BlockSpec((1, NC), lambda i: (0, 0)),
            pl.BlockSpec((1, NC), lambda i: (0, 0)),
            pl.BlockSpec((1, NC), lambda i: (0, 0)),
            pl.BlockSpec((1, NC), lambda i: (0, 0)),
        ],
        out_specs=pl.BlockSpec((1, H, W, NC), lambda i: (i, 0, 0, 0)),
        out_shape=jax.ShapeDtypeStruct((B, H, W, NC), jnp.float32),
        scratch_shapes=[pltpu.VMEM((10, W + 2, NC), jnp.bfloat16)],
    )(xt, oh, taps, w2, sums, sqs,
      bn_weight.reshape(1, NC), bn_bias.reshape(1, NC),
      conv_gamma_b.reshape(1, NC), conv_beta_b.reshape(1, NC))

    return jnp.transpose(out_nhwc, (0, 3, 1, 2))
